# baseline (speedup 1.0000x reference)
import functools

import jax
import jax.numpy as jnp
from jax import lax
from jax.experimental import pallas as pl
from jax.experimental.pallas import tpu as pltpu
from jax.experimental.pallas import tpu_sc as plsc

B = 4
N = 65536
C = 64
NC = 3
K = 16384
MID = 64
NB = 2048
TW = 128

_NUM_SC_CORES = 2
_NUM_SUBCORES = 16
_NW = _NUM_SC_CORES * _NUM_SUBCORES
_ROWS_PER_W = (B * K) // _NW
_CHUNK = 512


def _stage1_body(f_ref, p_ref, W1_ref, W2_ref, g_ref, be_ref, mu_ref, va_ref,
                 cls_ref, s_ref, tab_ref):
    f = f_ref[0]
    h = lax.dot_general(W1_ref[...], f, (((1,), (0,)), ((), ())),
                        preferred_element_type=jnp.float32)
    h = (h - mu_ref[...]) / jnp.sqrt(va_ref[...] + 1e-5) * g_ref[...] + be_ref[...]
    h = jnp.maximum(h, 0.0)
    cls = lax.dot_general(W2_ref[...], h, (((1,), (0,)), ((), ())),
                          preferred_element_type=jnp.float32)
    s = jax.nn.sigmoid(jnp.max(cls, axis=0, keepdims=True))
    cls_ref[0] = cls
    s_ref[0, 0] = s[0]
    tab_ref[0] = jnp.concatenate(
        [jnp.swapaxes(f, 0, 1), p_ref[0],
         jnp.zeros((NB, TW - C - 3), dtype=jnp.float32)], axis=1)


def _stage1(features, points, W1, W2, gamma1, beta1, mean1, var1):
    vec = pl.BlockSpec((C, 1), lambda b, n: (0, 0))
    return pl.pallas_call(
        _stage1_body,
        grid=(B, N // NB),
        in_specs=[
            pl.BlockSpec((1, C, NB), lambda b, n: (b, 0, n)),
            pl.BlockSpec((1, NB, 3), lambda b, n: (b, n, 0)),
            pl.BlockSpec((C, C), lambda b, n: (0, 0)),
            pl.BlockSpec((NC, C), lambda b, n: (0, 0)),
            vec, vec, vec, vec,
        ],
        out_specs=[
            pl.BlockSpec((1, NC, NB), lambda b, n: (b, 0, n)),
            pl.BlockSpec((1, 1, NB), lambda b, n: (b, 0, n)),
            pl.BlockSpec((1, NB, TW), lambda b, n: (b, n, 0)),
        ],
        out_shape=[
            jax.ShapeDtypeStruct((B, NC, N), jnp.float32),
            jax.ShapeDtypeStruct((B, 1, N), jnp.float32),
            jax.ShapeDtypeStruct((B, N, TW), jnp.float32),
        ],
    )(features, points, W1, W2,
      gamma1[:, None], beta1[:, None], mean1[:, None], var1[:, None])


_ROWS = N // 128


def _compare_exchange(val, idx, flatpos, asc, j):
    s = 1 << j
    if s < 128:
        vm = pltpu.roll(val, 128 - s, axis=1)
        vp = pltpu.roll(val, s, axis=1)
        im = pltpu.roll(idx, 128 - s, axis=1)
        ip = pltpu.roll(idx, s, axis=1)
    else:
        sr = s // 128
        vm = pltpu.roll(val, _ROWS - sr, axis=0)
        vp = pltpu.roll(val, sr, axis=0)
        im = pltpu.roll(idx, _ROWS - sr, axis=0)
        ip = pltpu.roll(idx, sr, axis=0)
    low = (flatpos & s) == 0
    pv = jnp.where(low, vm, vp)
    pi = jnp.where(low, im, ip)
    d = (val > pv) | ((val == pv) & (idx < pi))
    swap = asc == (d == low)
    return jnp.where(swap, pv, val), jnp.where(swap, pi, idx)


def _sort_body(s_ref, out_ref):
    row = lax.broadcasted_iota(jnp.int32, (_ROWS, 128), 0)
    lane = lax.broadcasted_iota(jnp.int32, (_ROWS, 128), 1)
    flatpos = row * 128 + lane
    val = [s_ref[b, 0, :].reshape(_ROWS, 128) for b in range(B)]
    idx = [flatpos for _ in range(B)]
    for k in range(1, 17):
        asc = ((flatpos >> k) & 1) == 1
        for j in range(k - 1, -1, -1):
            for b in range(B):
                val[b], idx[b] = _compare_exchange(val[b], idx[b], flatpos, asc, j)
    for b in range(B):
        out_ref[b, 0, :] = (idx[b][:K // 128, :] + b * N).reshape(K)


def _sort(scores):
    return pl.pallas_call(
        _sort_body,
        out_shape=jax.ShapeDtypeStruct((B, 1, K), jnp.int32),
    )(scores)


def _sc_gather_kernel(tab_hbm, idx_hbm, out_hbm, idx_v, row_v, sem):
    wid = lax.axis_index("s") * _NUM_SC_CORES + lax.axis_index("c")
    for t in range(_ROWS_PER_W // _CHUNK):
        base = wid * _ROWS_PER_W + t * _CHUNK
        pltpu.sync_copy(idx_hbm.at[pl.ds(base, _CHUNK)], idx_v)
        pltpu.async_copy(tab_hbm.at[idx_v], row_v, sem).wait()
        pltpu.sync_copy(row_v, out_hbm.at[pl.ds(base, _CHUNK)])


def _sc_gather(table, idx_flat):
    mesh = plsc.VectorSubcoreMesh(core_axis_name="c", subcore_axis_name="s")
    run = functools.partial(
        pl.kernel,
        mesh=mesh,
        out_type=jax.ShapeDtypeStruct((B * K, TW), jnp.float32),
        scratch_types=[
            pltpu.VMEM((_CHUNK,), jnp.int32),
            pltpu.VMEM((_CHUNK, TW), jnp.float32),
            pltpu.SemaphoreType.DMA,
        ],
    )(_sc_gather_kernel)
    return run(table, idx_flat)


def _stage2_body(gt_ref, W3_ref, W4_ref, g_ref, be_ref, mu_ref, va_ref,
                 pred_ref, orig_ref, off_ref):
    fg = gt_ref[0, :, 0:C]
    h = lax.dot_general(fg, W3_ref[...], (((1,), (1,)), ((), ())),
                        preferred_element_type=jnp.float32)
    h = (h - mu_ref[...]) / jnp.sqrt(va_ref[...] + 1e-5) * g_ref[...] + be_ref[...]
    h = jnp.maximum(h, 0.0)
    off = lax.dot_general(h, W4_ref[...], (((1,), (1,)), ((), ())),
                          preferred_element_type=jnp.float32)
    lane = lax.broadcasted_iota(jnp.int32, (NB, 3), 1)
    lim = jnp.where(lane < 2, 3.0, 2.0).astype(jnp.float32)
    limited = jnp.where(off > lim, lim, off)
    limited = jnp.where(limited < -lim, -lim, limited)
    orig = gt_ref[0, :, C:C + 3]
    pred_ref[0] = orig + limited
    orig_ref[0] = orig
    off_ref[0] = off


def _stage2(gath, W3, W4, gamma2, beta2, mean2, var2):
    vec = pl.BlockSpec((1, MID), lambda b, n: (0, 0))
    return pl.pallas_call(
        _stage2_body,
        grid=(B, K // NB),
        in_specs=[
            pl.BlockSpec((1, NB, TW), lambda b, n: (b, n, 0)),
            pl.BlockSpec((MID, C), lambda b, n: (0, 0)),
            pl.BlockSpec((3, MID), lambda b, n: (0, 0)),
            vec, vec, vec, vec,
        ],
        out_specs=[
            pl.BlockSpec((1, NB, 3), lambda b, n: (b, n, 0)),
            pl.BlockSpec((1, NB, 3), lambda b, n: (b, n, 0)),
            pl.BlockSpec((1, NB, 3), lambda b, n: (b, n, 0)),
        ],
        out_shape=[
            jax.ShapeDtypeStruct((B, K, 3), jnp.float32),
            jax.ShapeDtypeStruct((B, K, 3), jnp.float32),
            jax.ShapeDtypeStruct((B, K, 3), jnp.float32),
        ],
    )(gath, W3, W4,
      gamma2[None, :], beta2[None, :], mean2[None, :], var2[None, :])


def kernel(points, features, W1, gamma1, beta1, mean1, var1, W2,
           W3, gamma2, beta2, mean2, var2, W4):
    cls_preds, scores, table = _stage1(
        features, points, W1, W2, gamma1, beta1, mean1, var1)
    idx_glob = _sort(scores)
    gath = _sc_gather(table.reshape(B * N, TW), idx_glob.reshape(B * K))
    ctr_preds, ctr_origins, ctr_offsets = _stage2(
        gath.reshape(B, K, TW), W3, W4, gamma2, beta2, mean2, var2)
    return (ctr_preds, ctr_origins, ctr_offsets, cls_preds)


# speedup vs baseline: 1.0258x; 1.0258x over previous
import functools

import jax
import jax.numpy as jnp
from jax import lax
from jax.experimental import pallas as pl
from jax.experimental.pallas import tpu as pltpu
from jax.experimental.pallas import tpu_sc as plsc

B = 4
N = 65536
C = 64
NC = 3
K = 16384
MID = 64
NB = 2048
TW = 128

_NUM_SC_CORES = 2
_NUM_SUBCORES = 16
_NW = _NUM_SC_CORES * _NUM_SUBCORES
_ROWS_PER_W = (B * K) // _NW
_CHUNK = 512


def _stage1_body(f_ref, p_ref, W1_ref, W2_ref, g_ref, be_ref, mu_ref, va_ref,
                 cls_ref, s_ref, tab_ref):
    f = f_ref[0]
    h = lax.dot_general(W1_ref[...], f, (((1,), (0,)), ((), ())),
                        preferred_element_type=jnp.float32)
    h = (h - mu_ref[...]) / jnp.sqrt(va_ref[...] + 1e-5) * g_ref[...] + be_ref[...]
    h = jnp.maximum(h, 0.0)
    cls = lax.dot_general(W2_ref[...], h, (((1,), (0,)), ((), ())),
                          preferred_element_type=jnp.float32)
    s = jax.nn.sigmoid(jnp.max(cls, axis=0, keepdims=True))
    cls_ref[0] = cls
    s_ref[0, 0] = s[0]
    tab_ref[0] = jnp.concatenate(
        [jnp.swapaxes(f, 0, 1), p_ref[0],
         jnp.zeros((NB, TW - C - 3), dtype=jnp.float32)], axis=1)


def _stage1(features, points, W1, W2, gamma1, beta1, mean1, var1):
    vec = pl.BlockSpec((C, 1), lambda b, n: (0, 0))
    return pl.pallas_call(
        _stage1_body,
        grid=(B, N // NB),
        in_specs=[
            pl.BlockSpec((1, C, NB), lambda b, n: (b, 0, n)),
            pl.BlockSpec((1, NB, 3), lambda b, n: (b, n, 0)),
            pl.BlockSpec((C, C), lambda b, n: (0, 0)),
            pl.BlockSpec((NC, C), lambda b, n: (0, 0)),
            vec, vec, vec, vec,
        ],
        out_specs=[
            pl.BlockSpec((1, NC, NB), lambda b, n: (b, 0, n)),
            pl.BlockSpec((1, 1, NB), lambda b, n: (b, 0, n)),
            pl.BlockSpec((1, NB, TW), lambda b, n: (b, n, 0)),
        ],
        out_shape=[
            jax.ShapeDtypeStruct((B, NC, N), jnp.float32),
            jax.ShapeDtypeStruct((B, 1, N), jnp.float32),
            jax.ShapeDtypeStruct((B, N, TW), jnp.float32),
        ],
    )(features, points, W1, W2,
      gamma1[:, None], beta1[:, None], mean1[:, None], var1[:, None])


_ROWS = N // 128


def _compare_exchange(val, idx, flatpos, asc, j):
    s = 1 << j
    rows = val.shape[0]
    if s < 128:
        vm = pltpu.roll(val, 128 - s, axis=1)
        vp = pltpu.roll(val, s, axis=1)
        im = pltpu.roll(idx, 128 - s, axis=1)
        ip = pltpu.roll(idx, s, axis=1)
    else:
        sr = s // 128
        vm = pltpu.roll(val, rows - sr, axis=0)
        vp = pltpu.roll(val, sr, axis=0)
        im = pltpu.roll(idx, rows - sr, axis=0)
        ip = pltpu.roll(idx, sr, axis=0)
    low = (flatpos & s) == 0
    pv = jnp.where(low, vm, vp)
    pi = jnp.where(low, im, ip)
    d = (val > pv) | ((val == pv) & (idx < pi))
    if asc is True:
        swap = d == low
    elif asc is False:
        swap = d ^ low
    else:
        swap = asc == (d == low)
    return jnp.where(swap, pv, val), jnp.where(swap, pi, idx)


def _before(av, ai, bv, bi):
    return (av > bv) | ((av == bv) & (ai < bi))


_KR = K // 128


def _sort_body(s_ref, out_ref):
    row = lax.broadcasted_iota(jnp.int32, (_ROWS, 128), 0)
    lane = lax.broadcasted_iota(jnp.int32, (_ROWS, 128), 1)
    flatpos = row * 128 + lane
    val = [s_ref[b, 0, :].reshape(_ROWS, 128) for b in range(B)]
    idx = [flatpos for _ in range(B)]
    for k in range(1, 15):
        asc = ((flatpos >> k) & 1) == 1
        for j in range(k - 1, -1, -1):
            for b in range(B):
                val[b], idx[b] = _compare_exchange(val[b], idx[b], flatpos, asc, j)
    fpq = flatpos[:_KR, :]
    fin = []
    for b in range(B):
        cv = [val[b][i * _KR:(i + 1) * _KR] for i in range(4)]
        ci = [idx[b][i * _KR:(i + 1) * _KR] for i in range(4)]
        d01 = _before(cv[0], ci[0], cv[1], ci[1])
        m01 = (jnp.where(d01, cv[0], cv[1]), jnp.where(d01, ci[0], ci[1]))
        d23 = _before(cv[2], ci[2], cv[3], ci[3])
        m23 = (jnp.where(d23, cv[2], cv[3]), jnp.where(d23, ci[2], ci[3]))
        fin.append((m01, m23))
    for j in range(13, -1, -1):
        for b in range(B):
            m01, m23 = fin[b]
            fin[b] = (_compare_exchange(*m01, fpq, False, j),
                      _compare_exchange(*m23, fpq, True, j))
    last = []
    for b in range(B):
        m01, m23 = fin[b]
        d3 = _before(m01[0], m01[1], m23[0], m23[1])
        last.append((jnp.where(d3, m01[0], m23[0]), jnp.where(d3, m01[1], m23[1])))
    for j in range(13, -1, -1):
        for b in range(B):
            last[b] = _compare_exchange(*last[b], fpq, False, j)
    for b in range(B):
        out_ref[b, 0, :] = (last[b][1] + b * N).reshape(K)


def _sort(scores):
    return pl.pallas_call(
        _sort_body,
        out_shape=jax.ShapeDtypeStruct((B, 1, K), jnp.int32),
    )(scores)


def _sc_gather_kernel(tab_hbm, idx_hbm, out_hbm, idx_v, row_v, sem):
    wid = lax.axis_index("s") * _NUM_SC_CORES + lax.axis_index("c")
    for t in range(_ROWS_PER_W // _CHUNK):
        base = wid * _ROWS_PER_W + t * _CHUNK
        pltpu.sync_copy(idx_hbm.at[pl.ds(base, _CHUNK)], idx_v)
        pltpu.async_copy(tab_hbm.at[idx_v], row_v, sem).wait()
        pltpu.sync_copy(row_v, out_hbm.at[pl.ds(base, _CHUNK)])


def _sc_gather(table, idx_flat):
    mesh = plsc.VectorSubcoreMesh(core_axis_name="c", subcore_axis_name="s")
    run = functools.partial(
        pl.kernel,
        mesh=mesh,
        out_type=jax.ShapeDtypeStruct((B * K, TW), jnp.float32),
        scratch_types=[
            pltpu.VMEM((_CHUNK,), jnp.int32),
            pltpu.VMEM((_CHUNK, TW), jnp.float32),
            pltpu.SemaphoreType.DMA,
        ],
    )(_sc_gather_kernel)
    return run(table, idx_flat)


def _stage2_body(gt_ref, W3_ref, W4_ref, g_ref, be_ref, mu_ref, va_ref,
                 pred_ref, orig_ref, off_ref):
    fg = gt_ref[0, :, 0:C]
    h = lax.dot_general(fg, W3_ref[...], (((1,), (1,)), ((), ())),
                        preferred_element_type=jnp.float32)
    h = (h - mu_ref[...]) / jnp.sqrt(va_ref[...] + 1e-5) * g_ref[...] + be_ref[...]
    h = jnp.maximum(h, 0.0)
    off = lax.dot_general(h, W4_ref[...], (((1,), (1,)), ((), ())),
                          preferred_element_type=jnp.float32)
    lane = lax.broadcasted_iota(jnp.int32, (NB, 3), 1)
    lim = jnp.where(lane < 2, 3.0, 2.0).astype(jnp.float32)
    limited = jnp.where(off > lim, lim, off)
    limited = jnp.where(limited < -lim, -lim, limited)
    orig = gt_ref[0, :, C:C + 3]
    pred_ref[0] = orig + limited
    orig_ref[0] = orig
    off_ref[0] = off


def _stage2(gath, W3, W4, gamma2, beta2, mean2, var2):
    vec = pl.BlockSpec((1, MID), lambda b, n: (0, 0))
    return pl.pallas_call(
        _stage2_body,
        grid=(B, K // NB),
        in_specs=[
            pl.BlockSpec((1, NB, TW), lambda b, n: (b, n, 0)),
            pl.BlockSpec((MID, C), lambda b, n: (0, 0)),
            pl.BlockSpec((3, MID), lambda b, n: (0, 0)),
            vec, vec, vec, vec,
        ],
        out_specs=[
            pl.BlockSpec((1, NB, 3), lambda b, n: (b, n, 0)),
            pl.BlockSpec((1, NB, 3), lambda b, n: (b, n, 0)),
            pl.BlockSpec((1, NB, 3), lambda b, n: (b, n, 0)),
        ],
        out_shape=[
            jax.ShapeDtypeStruct((B, K, 3), jnp.float32),
            jax.ShapeDtypeStruct((B, K, 3), jnp.float32),
            jax.ShapeDtypeStruct((B, K, 3), jnp.float32),
        ],
    )(gath, W3, W4,
      gamma2[None, :], beta2[None, :], mean2[None, :], var2[None, :])


def kernel(points, features, W1, gamma1, beta1, mean1, var1, W2,
           W3, gamma2, beta2, mean2, var2, W4):
    cls_preds, scores, table = _stage1(
        features, points, W1, W2, gamma1, beta1, mean1, var1)
    idx_glob = _sort(scores)
    gath = _sc_gather(table.reshape(B * N, TW), idx_glob.reshape(B * K))
    ctr_preds, ctr_origins, ctr_offsets = _stage2(
        gath.reshape(B, K, TW), W3, W4, gamma2, beta2, mean2, var2)
    return (ctr_preds, ctr_origins, ctr_offsets, cls_preds)


# speedup vs baseline: 1.1312x; 1.1028x over previous
import functools

import jax
import jax.numpy as jnp
from jax import lax
from jax.experimental import pallas as pl
from jax.experimental.pallas import tpu as pltpu
from jax.experimental.pallas import tpu_sc as plsc

B = 4
N = 65536
C = 64
NC = 3
K = 16384
MID = 64
NB = 4096
TW = 128

_NUM_SC_CORES = 2
_NUM_SUBCORES = 16
_NW = _NUM_SC_CORES * _NUM_SUBCORES
_ROWS_PER_W = (B * K) // _NW
_CHUNK = 512


def _stage1_body(f_ref, p_ref, W1_ref, W2_ref, g_ref, be_ref, mu_ref, va_ref,
                 cls_ref, s_ref, tab_ref):
    f = f_ref[0]
    h = lax.dot_general(W1_ref[...], f, (((1,), (0,)), ((), ())),
                        preferred_element_type=jnp.float32)
    h = (h - mu_ref[...]) / jnp.sqrt(va_ref[...] + 1e-5) * g_ref[...] + be_ref[...]
    h = jnp.maximum(h, 0.0)
    cls = lax.dot_general(W2_ref[...], h, (((1,), (0,)), ((), ())),
                          preferred_element_type=jnp.float32)
    s = jax.nn.sigmoid(jnp.max(cls, axis=0, keepdims=True))
    cls_ref[0] = cls
    s_ref[0, 0] = s[0]
    tab_ref[0] = jnp.concatenate(
        [jnp.swapaxes(f, 0, 1), p_ref[0],
         jnp.zeros((NB, TW - C - 3), dtype=jnp.float32)], axis=1)


def _stage1(features, points, W1, W2, gamma1, beta1, mean1, var1):
    vec = pl.BlockSpec((C, 1), lambda b, n: (0, 0))
    return pl.pallas_call(
        _stage1_body,
        grid=(B, N // NB),
        in_specs=[
            pl.BlockSpec((1, C, NB), lambda b, n: (b, 0, n)),
            pl.BlockSpec((1, NB, 3), lambda b, n: (b, n, 0)),
            pl.BlockSpec((C, C), lambda b, n: (0, 0)),
            pl.BlockSpec((NC, C), lambda b, n: (0, 0)),
            vec, vec, vec, vec,
        ],
        out_specs=[
            pl.BlockSpec((1, NC, NB), lambda b, n: (b, 0, n)),
            pl.BlockSpec((1, 1, NB), lambda b, n: (b, 0, n)),
            pl.BlockSpec((1, NB, TW), lambda b, n: (b, n, 0)),
        ],
        out_shape=[
            jax.ShapeDtypeStruct((B, NC, N), jnp.float32),
            jax.ShapeDtypeStruct((B, 1, N), jnp.float32),
            jax.ShapeDtypeStruct((B, N, TW), jnp.float32),
        ],
    )(features, points, W1, W2,
      gamma1[:, None], beta1[:, None], mean1[:, None], var1[:, None])


_ROWS = N // 128


def _compare_exchange(val, idx, flatpos, asc, j):
    s = 1 << j
    rows = val.shape[0]
    if s < 128:
        vm = pltpu.roll(val, 128 - s, axis=1)
        vp = pltpu.roll(val, s, axis=1)
        im = pltpu.roll(idx, 128 - s, axis=1)
        ip = pltpu.roll(idx, s, axis=1)
    else:
        sr = s // 128
        vm = pltpu.roll(val, rows - sr, axis=0)
        vp = pltpu.roll(val, sr, axis=0)
        im = pltpu.roll(idx, rows - sr, axis=0)
        ip = pltpu.roll(idx, sr, axis=0)
    low = (flatpos & s) == 0
    pv = jnp.where(low, vm, vp)
    pi = jnp.where(low, im, ip)
    d = (val > pv) | ((val == pv) & (idx < pi))
    if asc is True:
        swap = d == low
    elif asc is False:
        swap = d ^ low
    else:
        swap = asc == (d == low)
    return jnp.where(swap, pv, val), jnp.where(swap, pi, idx)


def _before(av, ai, bv, bi):
    return (av > bv) | ((av == bv) & (ai < bi))


_KR = K // 128


def _sort_body(s_ref, out_ref):
    row = lax.broadcasted_iota(jnp.int32, (_ROWS, 128), 0)
    lane = lax.broadcasted_iota(jnp.int32, (_ROWS, 128), 1)
    flatpos = row * 128 + lane
    val = [s_ref[b, 0, :].reshape(_ROWS, 128) for b in range(B)]
    idx = [flatpos for _ in range(B)]
    for k in range(1, 15):
        asc = ((flatpos >> k) & 1) == 1
        for j in range(k - 1, -1, -1):
            for b in range(B):
                val[b], idx[b] = _compare_exchange(val[b], idx[b], flatpos, asc, j)
    fpq = flatpos[:_KR, :]
    fin = []
    for b in range(B):
        cv = [val[b][i * _KR:(i + 1) * _KR] for i in range(4)]
        ci = [idx[b][i * _KR:(i + 1) * _KR] for i in range(4)]
        d01 = _before(cv[0], ci[0], cv[1], ci[1])
        m01 = (jnp.where(d01, cv[0], cv[1]), jnp.where(d01, ci[0], ci[1]))
        d23 = _before(cv[2], ci[2], cv[3], ci[3])
        m23 = (jnp.where(d23, cv[2], cv[3]), jnp.where(d23, ci[2], ci[3]))
        fin.append((m01, m23))
    for j in range(13, -1, -1):
        for b in range(B):
            m01, m23 = fin[b]
            fin[b] = (_compare_exchange(*m01, fpq, False, j),
                      _compare_exchange(*m23, fpq, True, j))
    last = []
    for b in range(B):
        m01, m23 = fin[b]
        d3 = _before(m01[0], m01[1], m23[0], m23[1])
        last.append((jnp.where(d3, m01[0], m23[0]), jnp.where(d3, m01[1], m23[1])))
    for j in range(13, -1, -1):
        for b in range(B):
            last[b] = _compare_exchange(*last[b], fpq, False, j)
    for b in range(B):
        out_ref[b, 0, :] = (last[b][1] + b * N).reshape(K)


def _sort(scores):
    return pl.pallas_call(
        _sort_body,
        out_shape=jax.ShapeDtypeStruct((B, 1, K), jnp.int32),
    )(scores)


def _sc_gather_kernel(tab_hbm, idx_hbm, out_hbm, idx_v, row_v, sem):
    wid = lax.axis_index("s") * _NUM_SC_CORES + lax.axis_index("c")
    for t in range(_ROWS_PER_W // _CHUNK):
        base = wid * _ROWS_PER_W + t * _CHUNK
        pltpu.sync_copy(idx_hbm.at[pl.ds(base, _CHUNK)], idx_v)
        pltpu.async_copy(tab_hbm.at[idx_v], row_v, sem).wait()
        pltpu.sync_copy(row_v, out_hbm.at[pl.ds(base, _CHUNK)])


def _sc_gather(table, idx_flat):
    mesh = plsc.VectorSubcoreMesh(core_axis_name="c", subcore_axis_name="s")
    run = functools.partial(
        pl.kernel,
        mesh=mesh,
        out_type=jax.ShapeDtypeStruct((B * K, TW), jnp.float32),
        scratch_types=[
            pltpu.VMEM((_CHUNK,), jnp.int32),
            pltpu.VMEM((_CHUNK, TW), jnp.float32),
            pltpu.SemaphoreType.DMA,
        ],
    )(_sc_gather_kernel)
    return run(table, idx_flat)


def _stage2_body(gt_ref, W3_ref, W4_ref, g_ref, be_ref, mu_ref, va_ref,
                 pred_ref, orig_ref, off_ref):
    fg = gt_ref[0, :, 0:C]
    h = lax.dot_general(fg, W3_ref[...], (((1,), (1,)), ((), ())),
                        preferred_element_type=jnp.float32)
    h = (h - mu_ref[...]) / jnp.sqrt(va_ref[...] + 1e-5) * g_ref[...] + be_ref[...]
    h = jnp.maximum(h, 0.0)
    off = lax.dot_general(h, W4_ref[...], (((1,), (1,)), ((), ())),
                          preferred_element_type=jnp.float32)
    lane = lax.broadcasted_iota(jnp.int32, (NB, 3), 1)
    lim = jnp.where(lane < 2, 3.0, 2.0).astype(jnp.float32)
    limited = jnp.where(off > lim, lim, off)
    limited = jnp.where(limited < -lim, -lim, limited)
    orig = gt_ref[0, :, C:C + 3]
    pred_ref[0] = orig + limited
    orig_ref[0] = orig
    off_ref[0] = off


def _stage2(gath, W3, W4, gamma2, beta2, mean2, var2):
    vec = pl.BlockSpec((1, MID), lambda b, n: (0, 0))
    return pl.pallas_call(
        _stage2_body,
        grid=(B, K // NB),
        in_specs=[
            pl.BlockSpec((1, NB, TW), lambda b, n: (b, n, 0)),
            pl.BlockSpec((MID, C), lambda b, n: (0, 0)),
            pl.BlockSpec((3, MID), lambda b, n: (0, 0)),
            vec, vec, vec, vec,
        ],
        out_specs=[
            pl.BlockSpec((1, NB, 3), lambda b, n: (b, n, 0)),
            pl.BlockSpec((1, NB, 3), lambda b, n: (b, n, 0)),
            pl.BlockSpec((1, NB, 3), lambda b, n: (b, n, 0)),
        ],
        out_shape=[
            jax.ShapeDtypeStruct((B, K, 3), jnp.float32),
            jax.ShapeDtypeStruct((B, K, 3), jnp.float32),
            jax.ShapeDtypeStruct((B, K, 3), jnp.float32),
        ],
    )(gath, W3, W4,
      gamma2[None, :], beta2[None, :], mean2[None, :], var2[None, :])


def kernel(points, features, W1, gamma1, beta1, mean1, var1, W2,
           W3, gamma2, beta2, mean2, var2, W4):
    cls_preds, scores, table = _stage1(
        features, points, W1, W2, gamma1, beta1, mean1, var1)
    idx_glob = _sort(scores)
    gath = _sc_gather(table.reshape(B * N, TW), idx_glob.reshape(B * K))
    ctr_preds, ctr_origins, ctr_offsets = _stage2(
        gath.reshape(B, K, TW), W3, W4, gamma2, beta2, mean2, var2)
    return (ctr_preds, ctr_origins, ctr_offsets, cls_preds)


# speedup vs baseline: 1.1878x; 1.0500x over previous
import functools

import jax
import jax.numpy as jnp
from jax import lax
from jax.experimental import pallas as pl
from jax.experimental.pallas import tpu as pltpu
from jax.experimental.pallas import tpu_sc as plsc

B = 4
N = 65536
C = 64
NC = 3
K = 16384
MID = 64
NB = 8192
TW = 128

_NUM_SC_CORES = 2
_NUM_SUBCORES = 16
_NW = _NUM_SC_CORES * _NUM_SUBCORES
_ROWS_PER_W = (B * K) // _NW
_CHUNK = 512


def _stage1_body(f_ref, p_ref, W1_ref, W2_ref, g_ref, be_ref, mu_ref, va_ref,
                 cls_ref, s_ref, tab_ref):
    f = f_ref[0]
    h = lax.dot_general(W1_ref[...], f, (((1,), (0,)), ((), ())),
                        preferred_element_type=jnp.float32)
    h = (h - mu_ref[...]) / jnp.sqrt(va_ref[...] + 1e-5) * g_ref[...] + be_ref[...]
    h = jnp.maximum(h, 0.0)
    cls = lax.dot_general(W2_ref[...], h, (((1,), (0,)), ((), ())),
                          preferred_element_type=jnp.float32)
    s = jax.nn.sigmoid(jnp.max(cls, axis=0, keepdims=True))
    cls_ref[0] = cls
    s_ref[0, 0] = s[0]
    tab_ref[0] = jnp.concatenate(
        [jnp.swapaxes(f, 0, 1), p_ref[0],
         jnp.zeros((NB, TW - C - 3), dtype=jnp.float32)], axis=1)


def _stage1(features, points, W1, W2, gamma1, beta1, mean1, var1):
    vec = pl.BlockSpec((C, 1), lambda b, n: (0, 0))
    return pl.pallas_call(
        _stage1_body,
        grid=(B, N // NB),
        in_specs=[
            pl.BlockSpec((1, C, NB), lambda b, n: (b, 0, n)),
            pl.BlockSpec((1, NB, 3), lambda b, n: (b, n, 0)),
            pl.BlockSpec((C, C), lambda b, n: (0, 0)),
            pl.BlockSpec((NC, C), lambda b, n: (0, 0)),
            vec, vec, vec, vec,
        ],
        out_specs=[
            pl.BlockSpec((1, NC, NB), lambda b, n: (b, 0, n)),
            pl.BlockSpec((1, 1, NB), lambda b, n: (b, 0, n)),
            pl.BlockSpec((1, NB, TW), lambda b, n: (b, n, 0)),
        ],
        out_shape=[
            jax.ShapeDtypeStruct((B, NC, N), jnp.float32),
            jax.ShapeDtypeStruct((B, 1, N), jnp.float32),
            jax.ShapeDtypeStruct((B, N, TW), jnp.float32),
        ],
    )(features, points, W1, W2,
      gamma1[:, None], beta1[:, None], mean1[:, None], var1[:, None])


_ROWS = N // 128


def _compare_exchange(val, idx, flatpos, asc, j):
    s = 1 << j
    rows = val.shape[0]
    if s < 128:
        vm = pltpu.roll(val, 128 - s, axis=1)
        vp = pltpu.roll(val, s, axis=1)
        im = pltpu.roll(idx, 128 - s, axis=1)
        ip = pltpu.roll(idx, s, axis=1)
    else:
        sr = s // 128
        vm = pltpu.roll(val, rows - sr, axis=0)
        vp = pltpu.roll(val, sr, axis=0)
        im = pltpu.roll(idx, rows - sr, axis=0)
        ip = pltpu.roll(idx, sr, axis=0)
    low = (flatpos & s) == 0
    pv = jnp.where(low, vm, vp)
    pi = jnp.where(low, im, ip)
    d = (val > pv) | ((val == pv) & (idx < pi))
    if asc is True:
        swap = d == low
    elif asc is False:
        swap = d ^ low
    else:
        swap = asc == (d == low)
    return jnp.where(swap, pv, val), jnp.where(swap, pi, idx)


def _before(av, ai, bv, bi):
    return (av > bv) | ((av == bv) & (ai < bi))


_KR = K // 128


def _sort_body(s_ref, out_ref):
    row = lax.broadcasted_iota(jnp.int32, (_ROWS, 128), 0)
    lane = lax.broadcasted_iota(jnp.int32, (_ROWS, 128), 1)
    flatpos = row * 128 + lane
    val = [s_ref[b, 0, :].reshape(_ROWS, 128) for b in range(B)]
    idx = [flatpos for _ in range(B)]
    for k in range(1, 15):
        asc = ((flatpos >> k) & 1) == 1
        for j in range(k - 1, -1, -1):
            for b in range(B):
                val[b], idx[b] = _compare_exchange(val[b], idx[b], flatpos, asc, j)
    fpq = flatpos[:_KR, :]
    fin = []
    for b in range(B):
        cv = [val[b][i * _KR:(i + 1) * _KR] for i in range(4)]
        ci = [idx[b][i * _KR:(i + 1) * _KR] for i in range(4)]
        d01 = _before(cv[0], ci[0], cv[1], ci[1])
        m01 = (jnp.where(d01, cv[0], cv[1]), jnp.where(d01, ci[0], ci[1]))
        d23 = _before(cv[2], ci[2], cv[3], ci[3])
        m23 = (jnp.where(d23, cv[2], cv[3]), jnp.where(d23, ci[2], ci[3]))
        fin.append((m01, m23))
    for j in range(13, -1, -1):
        for b in range(B):
            m01, m23 = fin[b]
            fin[b] = (_compare_exchange(*m01, fpq, False, j),
                      _compare_exchange(*m23, fpq, True, j))
    last = []
    for b in range(B):
        m01, m23 = fin[b]
        d3 = _before(m01[0], m01[1], m23[0], m23[1])
        last.append((jnp.where(d3, m01[0], m23[0]), jnp.where(d3, m01[1], m23[1])))
    for j in range(13, -1, -1):
        for b in range(B):
            last[b] = _compare_exchange(*last[b], fpq, False, j)
    for b in range(B):
        out_ref[b, 0, :] = (last[b][1] + b * N).reshape(K)


def _sort(scores):
    return pl.pallas_call(
        _sort_body,
        out_shape=jax.ShapeDtypeStruct((B, 1, K), jnp.int32),
    )(scores)


def _sc_gather_kernel(tab_hbm, idx_hbm, out_hbm, idx_v, row_v, sem):
    wid = lax.axis_index("s") * _NUM_SC_CORES + lax.axis_index("c")
    for t in range(_ROWS_PER_W // _CHUNK):
        base = wid * _ROWS_PER_W + t * _CHUNK
        pltpu.sync_copy(idx_hbm.at[pl.ds(base, _CHUNK)], idx_v)
        pltpu.async_copy(tab_hbm.at[idx_v], row_v, sem).wait()
        pltpu.sync_copy(row_v, out_hbm.at[pl.ds(base, _CHUNK)])


def _sc_gather(table, idx_flat):
    mesh = plsc.VectorSubcoreMesh(core_axis_name="c", subcore_axis_name="s")
    run = functools.partial(
        pl.kernel,
        mesh=mesh,
        out_type=jax.ShapeDtypeStruct((B * K, TW), jnp.float32),
        scratch_types=[
            pltpu.VMEM((_CHUNK,), jnp.int32),
            pltpu.VMEM((_CHUNK, TW), jnp.float32),
            pltpu.SemaphoreType.DMA,
        ],
    )(_sc_gather_kernel)
    return run(table, idx_flat)


def _stage2_body(gt_ref, W3_ref, W4_ref, g_ref, be_ref, mu_ref, va_ref,
                 pred_ref, orig_ref, off_ref):
    fg = gt_ref[0, :, 0:C]
    h = lax.dot_general(fg, W3_ref[...], (((1,), (1,)), ((), ())),
                        preferred_element_type=jnp.float32)
    h = (h - mu_ref[...]) / jnp.sqrt(va_ref[...] + 1e-5) * g_ref[...] + be_ref[...]
    h = jnp.maximum(h, 0.0)
    off = lax.dot_general(h, W4_ref[...], (((1,), (1,)), ((), ())),
                          preferred_element_type=jnp.float32)
    lane = lax.broadcasted_iota(jnp.int32, (NB, 3), 1)
    lim = jnp.where(lane < 2, 3.0, 2.0).astype(jnp.float32)
    limited = jnp.where(off > lim, lim, off)
    limited = jnp.where(limited < -lim, -lim, limited)
    orig = gt_ref[0, :, C:C + 3]
    pred_ref[0] = orig + limited
    orig_ref[0] = orig
    off_ref[0] = off


def _stage2(gath, W3, W4, gamma2, beta2, mean2, var2):
    vec = pl.BlockSpec((1, MID), lambda b, n: (0, 0))
    return pl.pallas_call(
        _stage2_body,
        grid=(B, K // NB),
        in_specs=[
            pl.BlockSpec((1, NB, TW), lambda b, n: (b, n, 0)),
            pl.BlockSpec((MID, C), lambda b, n: (0, 0)),
            pl.BlockSpec((3, MID), lambda b, n: (0, 0)),
            vec, vec, vec, vec,
        ],
        out_specs=[
            pl.BlockSpec((1, NB, 3), lambda b, n: (b, n, 0)),
            pl.BlockSpec((1, NB, 3), lambda b, n: (b, n, 0)),
            pl.BlockSpec((1, NB, 3), lambda b, n: (b, n, 0)),
        ],
        out_shape=[
            jax.ShapeDtypeStruct((B, K, 3), jnp.float32),
            jax.ShapeDtypeStruct((B, K, 3), jnp.float32),
            jax.ShapeDtypeStruct((B, K, 3), jnp.float32),
        ],
    )(gath, W3, W4,
      gamma2[None, :], beta2[None, :], mean2[None, :], var2[None, :])


def kernel(points, features, W1, gamma1, beta1, mean1, var1, W2,
           W3, gamma2, beta2, mean2, var2, W4):
    cls_preds, scores, table = _stage1(
        features, points, W1, W2, gamma1, beta1, mean1, var1)
    idx_glob = _sort(scores)
    gath = _sc_gather(table.reshape(B * N, TW), idx_glob.reshape(B * K))
    ctr_preds, ctr_origins, ctr_offsets = _stage2(
        gath.reshape(B, K, TW), W3, W4, gamma2, beta2, mean2, var2)
    return (ctr_preds, ctr_origins, ctr_offsets, cls_preds)
